# Initial kernel scaffold; baseline (speedup 1.0000x reference)
import jax
import jax.numpy as jnp
from jax.experimental import pallas as pl


def kernel(features, edge_index, W1, b1, W2, b2, W3, b3):
    raise NotImplementedError("write your pallas kernel here")


# speedup vs baseline: 7.1334x; 7.1334x over previous
import functools

import jax
import jax.numpy as jnp
from jax import lax
from jax.experimental import pallas as pl
from jax.experimental.pallas import tpu as pltpu
from jax.experimental.pallas import tpu_sc as plsc

_N = 10000
_E = 320000
_NC = 2
_NS = 16
_NW = _NC * _NS
_EPT = _E // _NW
_K = 80
_NCH = _EPT // _K
_NP = 10240
_RPT = _NP // _NS
_R = 1000

_mesh = plsc.VectorSubcoreMesh(core_axis_name="c", subcore_axis_name="s")


@functools.partial(
    pl.kernel,
    mesh=_mesh,
    out_type=jax.ShapeDtypeStruct((_NC, _NP, 16), jnp.float32),
    scratch_types=[
        pltpu.VMEM((_NCH, _K), jnp.int32),
        pltpu.VMEM((_K, 16), jnp.float32),
        pltpu.VMEM_SHARED((_NP, 16), jnp.float32),
    ],
)
def _deg_kernel(dst_hbm, ones_hbm, zeros_hbm, out_hbm, dstv, onesv, shared):
    c = lax.axis_index("c")
    s = lax.axis_index("s")
    w = c * _NS + s
    pltpu.sync_copy(dst_hbm.at[w], dstv)
    pltpu.sync_copy(ones_hbm, onesv)
    pltpu.sync_copy(zeros_hbm, shared.at[pl.ds(s * _RPT, _RPT)])
    plsc.subcore_barrier()

    def body(j, carry):
        pltpu.sync_copy(onesv, shared.at[dstv.at[j]], add=True)
        return carry

    lax.fori_loop(0, _NCH, body, 0)
    plsc.subcore_barrier()
    pltpu.sync_copy(shared.at[pl.ds(s * _RPT, _RPT)],
                    out_hbm.at[c, pl.ds(s * _RPT, _RPT)])


def _make_agg(d):
    @functools.partial(
        pl.kernel,
        mesh=_mesh,
        out_type=jax.ShapeDtypeStruct((_NC, _NP, d), jnp.float32),
        scratch_types=[
            pltpu.VMEM((_NCH, _K), jnp.int32),
            pltpu.VMEM((_NCH, _K), jnp.int32),
            pltpu.VMEM((_K, d), jnp.float32),
            pltpu.VMEM_SHARED((_NP, d), jnp.float32),
            pltpu.SemaphoreType.DMA,
        ],
    )
    def agg(y_hbm, src_hbm, dst_hbm, zeros_hbm, out_hbm,
            srcv, dstv, buf, shared, sem):
        c = lax.axis_index("c")
        s = lax.axis_index("s")
        w = c * _NS + s
        pltpu.sync_copy(src_hbm.at[w], srcv)
        pltpu.sync_copy(dst_hbm.at[w], dstv)
        pltpu.sync_copy(zeros_hbm, shared.at[pl.ds(s * _RPT, _RPT)])
        plsc.subcore_barrier()

        def body(j, carry):
            pltpu.async_copy(y_hbm.at[srcv.at[j]], buf, sem).wait()
            pltpu.sync_copy(buf, shared.at[dstv.at[j]], add=True)
            return carry

        lax.fori_loop(0, _NCH, body, 0)
        plsc.subcore_barrier()
        pltpu.sync_copy(shared.at[pl.ds(s * _RPT, _RPT)],
                        out_hbm.at[c, pl.ds(s * _RPT, _RPT)])

    return agg


_agg128 = _make_agg(128)


def _tc1_body(x_ref, w_ref, degp_ref, y_ref, norm_ref):
    dp = degp_ref[...]
    deg = dp[0, :, 0:1] + dp[1, :, 0:1]
    norm = jnp.where(deg > 0, lax.rsqrt(jnp.maximum(deg, 1.0)), 0.0)
    y = norm * jnp.dot(x_ref[...], w_ref[...],
                       preferred_element_type=jnp.float32)
    y_ref[...] = y
    norm_ref[...] = jnp.broadcast_to(norm, norm_ref.shape)


def _tc_mid_body(p_ref, norm_ref, b_ref, w_ref, y_ref):
    a = p_ref[...]
    nb = norm_ref[...]
    h = (a[0] + a[1]) * nb + b_ref[...]
    y_ref[...] = nb[:, 0:1] * jnp.dot(h, w_ref[...],
                                      preferred_element_type=jnp.float32)


def _tc_scale_body(p_ref, norm_ref, b_ref, y_ref):
    a = p_ref[...]
    nb = norm_ref[...]
    y_ref[...] = nb * ((a[0] + a[1]) * nb + b_ref[...])


def _tc_final_body(p_ref, norm_ref, b_ref, w_ref, out_ref):
    a = p_ref[...]
    o = jnp.dot((a[0] + a[1]) * norm_ref[...], w_ref[...],
                preferred_element_type=jnp.float32) + b_ref[...]
    m = jnp.max(o, axis=1, keepdims=True)
    ls = jnp.log(jnp.sum(jnp.exp(o - m), axis=1, keepdims=True))
    out_ref[...] = o - m - ls


def _tc1(x, w1, degp):
    return pl.pallas_call(
        _tc1_body,
        grid=(_N // _R,),
        in_specs=[
            pl.BlockSpec((_R, 128), lambda i: (i, 0)),
            pl.BlockSpec((128, 128), lambda i: (0, 0)),
            pl.BlockSpec((_NC, _R, 16), lambda i: (0, i, 0)),
        ],
        out_specs=[
            pl.BlockSpec((_R, 128), lambda i: (i, 0)),
            pl.BlockSpec((_R, 128), lambda i: (i, 0)),
        ],
        out_shape=[
            jax.ShapeDtypeStruct((_N, 128), jnp.float32),
            jax.ShapeDtypeStruct((_N, 128), jnp.float32),
        ],
    )(x, w1, degp)


def _tc_mid(p, normb, b, w, d_out):
    return pl.pallas_call(
        _tc_mid_body,
        grid=(_N // _R,),
        in_specs=[
            pl.BlockSpec((_NC, _R, 128), lambda i: (0, i, 0)),
            pl.BlockSpec((_R, 128), lambda i: (i, 0)),
            pl.BlockSpec((1, 128), lambda i: (0, 0)),
            pl.BlockSpec((128, d_out), lambda i: (0, 0)),
        ],
        out_specs=pl.BlockSpec((_R, d_out), lambda i: (i, 0)),
        out_shape=jax.ShapeDtypeStruct((_N, d_out), jnp.float32),
    )(p, normb, b, w)


def _tc_scale(p, normb, b):
    return pl.pallas_call(
        _tc_scale_body,
        grid=(_N // _R,),
        in_specs=[
            pl.BlockSpec((_NC, _R, 128), lambda i: (0, i, 0)),
            pl.BlockSpec((_R, 128), lambda i: (i, 0)),
            pl.BlockSpec((1, 128), lambda i: (0, 0)),
        ],
        out_specs=pl.BlockSpec((_R, 128), lambda i: (i, 0)),
        out_shape=jax.ShapeDtypeStruct((_N, 128), jnp.float32),
    )(p, normb, b)


def _tc_final(p, normb, b, w):
    return pl.pallas_call(
        _tc_final_body,
        grid=(_N // _R,),
        in_specs=[
            pl.BlockSpec((_NC, _R, 128), lambda i: (0, i, 0)),
            pl.BlockSpec((_R, 128), lambda i: (i, 0)),
            pl.BlockSpec((1, 64), lambda i: (0, 0)),
            pl.BlockSpec((128, 64), lambda i: (0, 0)),
        ],
        out_specs=pl.BlockSpec((_R, 64), lambda i: (i, 0)),
        out_shape=jax.ShapeDtypeStruct((_N, 64), jnp.float32),
    )(p, normb, b, w)


def kernel(features, edge_index, W1, b1, W2, b2, W3, b3):
    src = edge_index[0].reshape(_NW, _NCH, _K)
    dst = edge_index[1].reshape(_NW, _NCH, _K)
    ones16 = jnp.ones((_K, 16), jnp.float32)
    z16 = jnp.zeros((_RPT, 16), jnp.float32)
    z128 = jnp.zeros((_RPT, 128), jnp.float32)

    degp = _deg_kernel(dst, ones16, z16)
    y1, normb = _tc1(features, W1, degp)
    p1 = _agg128(y1, src, dst, z128)
    y2 = _tc_mid(p1, normb, b1.reshape(1, 128), W2, 128)
    p2 = _agg128(y2, src, dst, z128)
    y3 = _tc_scale(p2, normb, b2.reshape(1, 128))
    p3 = _agg128(y3, src, dst, z128)
    return _tc_final(p3, normb, b3.reshape(1, 64), W3)


# speedup vs baseline: 10.4836x; 1.4697x over previous
import functools

import jax
import jax.numpy as jnp
from jax import lax
from jax.experimental import pallas as pl
from jax.experimental.pallas import tpu as pltpu
from jax.experimental.pallas import tpu_sc as plsc

_N = 10000
_E = 320000
_NC = 2
_NS = 16
_NW = _NC * _NS
_EPT = _E // _NW
_K = 80
_NCH = _EPT // _K
_NPAN = 5
_CPP = _NCH // _NPAN
_NP = 10240
_RPT = _NP // _NS
_R = 1000

_mesh = plsc.VectorSubcoreMesh(core_axis_name="c", subcore_axis_name="s")


@functools.partial(
    pl.kernel,
    mesh=_mesh,
    out_type=jax.ShapeDtypeStruct((_NC, _NP, 16), jnp.float32),
    scratch_types=[
        pltpu.VMEM((_NCH, _K), jnp.int32),
        pltpu.VMEM((_K, 16), jnp.float32),
        pltpu.VMEM_SHARED((_NP, 16), jnp.float32),
    ],
)
def _deg_kernel(dst_hbm, ones_hbm, zeros_hbm, out_hbm, dstv, onesv, shared):
    c = lax.axis_index("c")
    s = lax.axis_index("s")
    w = c * _NS + s
    pltpu.sync_copy(dst_hbm.at[w], dstv)
    pltpu.sync_copy(ones_hbm, onesv)
    pltpu.sync_copy(zeros_hbm, shared.at[pl.ds(s * _RPT, _RPT)])
    plsc.subcore_barrier()

    def body(j, carry):
        pltpu.sync_copy(onesv, shared.at[dstv.at[j]], add=True)
        return carry

    lax.fori_loop(0, _NCH, body, 0)
    plsc.subcore_barrier()
    pltpu.sync_copy(shared.at[pl.ds(s * _RPT, _RPT)],
                    out_hbm.at[c, pl.ds(s * _RPT, _RPT)])


def _make_agg(d):
    @functools.partial(
        pl.kernel,
        mesh=_mesh,
        out_type=jax.ShapeDtypeStruct((_NC, _NP, d), jnp.float32),
        scratch_types=[
            pltpu.VMEM((_CPP, _K), jnp.int32),
            pltpu.VMEM((_CPP, _K), jnp.int32),
            pltpu.VMEM((_K, d), jnp.float32),
            pltpu.VMEM((_K, d), jnp.float32),
            pltpu.VMEM_SHARED((_NP, d), jnp.float32),
            pltpu.SemaphoreType.DMA,
            pltpu.SemaphoreType.DMA,
        ],
    )
    def agg(y_hbm, src_hbm, dst_hbm, zeros_hbm, out_hbm,
            srcp, dstp, buf0, buf1, shared, sem0, sem1):
        c = lax.axis_index("c")
        s = lax.axis_index("s")
        w = c * _NS + s
        pltpu.sync_copy(zeros_hbm, shared.at[pl.ds(s * _RPT, _RPT)])
        plsc.subcore_barrier()

        @pl.loop(0, _NPAN)
        def _panel(p):
            pltpu.sync_copy(src_hbm.at[w, p], srcp)
            pltpu.sync_copy(dst_hbm.at[w, p], dstp)
            pltpu.async_copy(y_hbm.at[srcp.at[0]], buf0, sem0)

            @pl.loop(0, _CPP - 1, step=2)
            def _pipe(i):
                pltpu.async_copy(y_hbm.at[srcp.at[i + 1]], buf1, sem1)
                pltpu.make_async_copy(y_hbm.at[srcp.at[i]], buf0, sem0).wait()
                pltpu.sync_copy(buf0, shared.at[dstp.at[i]], add=True)
                pltpu.async_copy(y_hbm.at[srcp.at[i + 2]], buf0, sem0)
                pltpu.make_async_copy(y_hbm.at[srcp.at[i + 1]], buf1, sem1).wait()
                pltpu.sync_copy(buf1, shared.at[dstp.at[i + 1]], add=True)

            pltpu.make_async_copy(y_hbm.at[srcp.at[_CPP - 1]], buf0, sem0).wait()
            pltpu.sync_copy(buf0, shared.at[dstp.at[_CPP - 1]], add=True)

        plsc.subcore_barrier()
        pltpu.sync_copy(shared.at[pl.ds(s * _RPT, _RPT)],
                        out_hbm.at[c, pl.ds(s * _RPT, _RPT)])

    return agg


_agg128 = _make_agg(128)


def _tc1_body(x_ref, w_ref, degp_ref, y_ref, norm_ref):
    dp = degp_ref[...]
    deg = dp[0, :, 0:1] + dp[1, :, 0:1]
    norm = jnp.where(deg > 0, lax.rsqrt(jnp.maximum(deg, 1.0)), 0.0)
    y = norm * jnp.dot(x_ref[...], w_ref[...],
                       preferred_element_type=jnp.float32)
    y_ref[...] = y
    norm_ref[...] = jnp.broadcast_to(norm, norm_ref.shape)


def _tc_mid_body(p_ref, norm_ref, b_ref, w_ref, y_ref):
    a = p_ref[...]
    nb = norm_ref[...]
    h = (a[0] + a[1]) * nb + b_ref[...]
    y_ref[...] = nb[:, 0:1] * jnp.dot(h, w_ref[...],
                                      preferred_element_type=jnp.float32)


def _tc_scale_body(p_ref, norm_ref, b_ref, y_ref):
    a = p_ref[...]
    nb = norm_ref[...]
    y_ref[...] = nb * ((a[0] + a[1]) * nb + b_ref[...])


def _tc_final_body(p_ref, norm_ref, b_ref, w_ref, out_ref):
    a = p_ref[...]
    o = jnp.dot((a[0] + a[1]) * norm_ref[...], w_ref[...],
                preferred_element_type=jnp.float32) + b_ref[...]
    m = jnp.max(o, axis=1, keepdims=True)
    ls = jnp.log(jnp.sum(jnp.exp(o - m), axis=1, keepdims=True))
    out_ref[...] = o - m - ls


def _tc1(x, w1, degp):
    return pl.pallas_call(
        _tc1_body,
        grid=(_N // _R,),
        in_specs=[
            pl.BlockSpec((_R, 128), lambda i: (i, 0)),
            pl.BlockSpec((128, 128), lambda i: (0, 0)),
            pl.BlockSpec((_NC, _R, 16), lambda i: (0, i, 0)),
        ],
        out_specs=[
            pl.BlockSpec((_R, 128), lambda i: (i, 0)),
            pl.BlockSpec((_R, 128), lambda i: (i, 0)),
        ],
        out_shape=[
            jax.ShapeDtypeStruct((_N, 128), jnp.float32),
            jax.ShapeDtypeStruct((_N, 128), jnp.float32),
        ],
    )(x, w1, degp)


def _tc_mid(p, normb, b, w, d_out):
    return pl.pallas_call(
        _tc_mid_body,
        grid=(_N // _R,),
        in_specs=[
            pl.BlockSpec((_NC, _R, 128), lambda i: (0, i, 0)),
            pl.BlockSpec((_R, 128), lambda i: (i, 0)),
            pl.BlockSpec((1, 128), lambda i: (0, 0)),
            pl.BlockSpec((128, d_out), lambda i: (0, 0)),
        ],
        out_specs=pl.BlockSpec((_R, d_out), lambda i: (i, 0)),
        out_shape=jax.ShapeDtypeStruct((_N, d_out), jnp.float32),
    )(p, normb, b, w)


def _tc_scale(p, normb, b):
    return pl.pallas_call(
        _tc_scale_body,
        grid=(_N // _R,),
        in_specs=[
            pl.BlockSpec((_NC, _R, 128), lambda i: (0, i, 0)),
            pl.BlockSpec((_R, 128), lambda i: (i, 0)),
            pl.BlockSpec((1, 128), lambda i: (0, 0)),
        ],
        out_specs=pl.BlockSpec((_R, 128), lambda i: (i, 0)),
        out_shape=jax.ShapeDtypeStruct((_N, 128), jnp.float32),
    )(p, normb, b)


def _tc_final(p, normb, b, w):
    return pl.pallas_call(
        _tc_final_body,
        grid=(_N // _R,),
        in_specs=[
            pl.BlockSpec((_NC, _R, 128), lambda i: (0, i, 0)),
            pl.BlockSpec((_R, 128), lambda i: (i, 0)),
            pl.BlockSpec((1, 64), lambda i: (0, 0)),
            pl.BlockSpec((128, 64), lambda i: (0, 0)),
        ],
        out_specs=pl.BlockSpec((_R, 64), lambda i: (i, 0)),
        out_shape=jax.ShapeDtypeStruct((_N, 64), jnp.float32),
    )(p, normb, b, w)


def kernel(features, edge_index, W1, b1, W2, b2, W3, b3):
    src = edge_index[0].reshape(_NW, _NPAN, _CPP, _K)
    dst = edge_index[1].reshape(_NW, _NPAN, _CPP, _K)
    dst2 = edge_index[1].reshape(_NW, _NCH, _K)
    ones16 = jnp.ones((_K, 16), jnp.float32)
    z16 = jnp.zeros((_RPT, 16), jnp.float32)
    z128 = jnp.zeros((_RPT, 128), jnp.float32)

    degp = _deg_kernel(dst2, ones16, z16)
    y1, normb = _tc1(features, W1, degp)
    p1 = _agg128(y1, src, dst, z128)
    y2 = _tc_mid(p1, normb, b1.reshape(1, 128), W2, 128)
    p2 = _agg128(y2, src, dst, z128)
    y3 = _tc_scale(p2, normb, b2.reshape(1, 128))
    p3 = _agg128(y3, src, dst, z128)
    return _tc_final(p3, normb, b3.reshape(1, 64), W3)


# speedup vs baseline: 10.5047x; 1.0020x over previous
import functools

import jax
import jax.numpy as jnp
from jax import lax
from jax.experimental import pallas as pl
from jax.experimental.pallas import tpu as pltpu
from jax.experimental.pallas import tpu_sc as plsc

_N = 10000
_E = 320000
_NC = 2
_NS = 16
_NW = _NC * _NS
_EPT = _E // _NW
_K = 80
_NCH = _EPT // _K
_NPAN = 5
_CPP = _NCH // _NPAN
_NP = 10240
_RPT = _NP // _NS
_R = 1000

_mesh = plsc.VectorSubcoreMesh(core_axis_name="c", subcore_axis_name="s")


@functools.partial(
    pl.kernel,
    mesh=_mesh,
    out_type=jax.ShapeDtypeStruct((_NC, _NP, 16), jnp.float32),
    scratch_types=[
        pltpu.VMEM((_NCH, _K), jnp.int32),
        pltpu.VMEM((_K, 16), jnp.float32),
        pltpu.VMEM_SHARED((_NP, 16), jnp.float32),
    ],
)
def _deg_kernel(dst_hbm, ones_hbm, zeros_hbm, out_hbm, dstv, onesv, shared):
    c = lax.axis_index("c")
    s = lax.axis_index("s")
    w = c * _NS + s
    pltpu.sync_copy(dst_hbm.at[w], dstv)
    pltpu.sync_copy(ones_hbm, onesv)
    pltpu.sync_copy(zeros_hbm, shared.at[pl.ds(s * _RPT, _RPT)])
    plsc.subcore_barrier()

    def body(j, carry):
        pltpu.sync_copy(onesv, shared.at[dstv.at[j]], add=True)
        return carry

    lax.fori_loop(0, _NCH, body, 0)
    plsc.subcore_barrier()
    pltpu.sync_copy(shared.at[pl.ds(s * _RPT, _RPT)],
                    out_hbm.at[c, pl.ds(s * _RPT, _RPT)])


def _make_agg(d):
    @functools.partial(
        pl.kernel,
        mesh=_mesh,
        out_type=jax.ShapeDtypeStruct((_NC, _NP, d), jnp.float32),
        scratch_types=[
            pltpu.VMEM((_CPP, _K), jnp.int32),
            pltpu.VMEM((_CPP, _K), jnp.int32),
            pltpu.VMEM((4, _K, d), jnp.float32),
            pltpu.VMEM_SHARED((_NP, d), jnp.float32),
            pltpu.SemaphoreType.DMA,
            pltpu.SemaphoreType.DMA,
            pltpu.SemaphoreType.DMA,
            pltpu.SemaphoreType.DMA,
            pltpu.SemaphoreType.DMA,
            pltpu.SemaphoreType.DMA,
            pltpu.SemaphoreType.DMA,
            pltpu.SemaphoreType.DMA,
        ],
    )
    def agg(y_hbm, src_hbm, dst_hbm, zeros_hbm, out_hbm,
            srcp, dstp, bufs, shared,
            g0, g1, g2, g3, s0, s1, s2, s3):
        c = lax.axis_index("c")
        s = lax.axis_index("s")
        w = c * _NS + s
        gsem = (g0, g1, g2, g3)
        ssem = (s0, s1, s2, s3)
        pltpu.sync_copy(zeros_hbm, shared.at[pl.ds(s * _RPT, _RPT)])
        plsc.subcore_barrier()

        @pl.loop(0, _NPAN)
        def _panel(p):
            pltpu.sync_copy(src_hbm.at[w, p], srcp)
            pltpu.sync_copy(dst_hbm.at[w, p], dstp)
            for b in range(4):
                pltpu.async_copy(y_hbm.at[srcp.at[b]], bufs.at[b], gsem[b])
            for b in range(4):
                pltpu.make_async_copy(
                    y_hbm.at[srcp.at[b]], bufs.at[b], gsem[b]).wait()
                pltpu.async_copy(bufs.at[b], shared.at[dstp.at[b]],
                                 ssem[b], add=True)

            @pl.loop(4, _CPP - 1, step=4)
            def _pipe(i):
                for b in range(4):
                    pltpu.make_async_copy(
                        bufs.at[b], shared.at[dstp.at[i - 4 + b]],
                        ssem[b]).wait()
                    pltpu.async_copy(y_hbm.at[srcp.at[i + b]], bufs.at[b],
                                     gsem[b])
                for b in range(4):
                    pltpu.make_async_copy(
                        y_hbm.at[srcp.at[i + b]], bufs.at[b], gsem[b]).wait()
                    pltpu.async_copy(bufs.at[b], shared.at[dstp.at[i + b]],
                                     ssem[b], add=True)

            for b in range(4):
                pltpu.make_async_copy(
                    bufs.at[b], shared.at[dstp.at[_CPP - 5 + b]],
                    ssem[b]).wait()
            pltpu.async_copy(y_hbm.at[srcp.at[_CPP - 1]], bufs.at[0], gsem[0])
            pltpu.make_async_copy(
                y_hbm.at[srcp.at[_CPP - 1]], bufs.at[0], gsem[0]).wait()
            pltpu.sync_copy(bufs.at[0], shared.at[dstp.at[_CPP - 1]], add=True)

        plsc.subcore_barrier()
        pltpu.sync_copy(shared.at[pl.ds(s * _RPT, _RPT)],
                        out_hbm.at[c, pl.ds(s * _RPT, _RPT)])

    return agg


_agg128 = _make_agg(128)


def _tc_mm_body(x_ref, w_ref, u_ref):
    u_ref[...] = jnp.dot(x_ref[...], w_ref[...],
                         preferred_element_type=jnp.float32)


def _tc_norm_body(u_ref, degp_ref, y_ref, norm_ref):
    dp = degp_ref[...]
    deg = dp[0, :, 0:1] + dp[1, :, 0:1]
    norm = jnp.where(deg > 0, lax.rsqrt(jnp.maximum(deg, 1.0)), 0.0)
    y_ref[...] = norm * u_ref[...]
    norm_ref[...] = jnp.broadcast_to(norm, norm_ref.shape)


def _tc_mid_body(p_ref, norm_ref, b_ref, w_ref, y_ref):
    a = p_ref[...]
    nb = norm_ref[...]
    h = (a[0] + a[1]) * nb + b_ref[...]
    y_ref[...] = nb[:, 0:1] * jnp.dot(h, w_ref[...],
                                      preferred_element_type=jnp.float32)


def _tc_scale_body(p_ref, norm_ref, b_ref, y_ref):
    a = p_ref[...]
    nb = norm_ref[...]
    y_ref[...] = nb * ((a[0] + a[1]) * nb + b_ref[...])


def _tc_final_body(p_ref, norm_ref, b_ref, w_ref, out_ref):
    a = p_ref[...]
    o = jnp.dot((a[0] + a[1]) * norm_ref[...], w_ref[...],
                preferred_element_type=jnp.float32) + b_ref[...]
    m = jnp.max(o, axis=1, keepdims=True)
    ls = jnp.log(jnp.sum(jnp.exp(o - m), axis=1, keepdims=True))
    out_ref[...] = o - m - ls


def _tc_mm(x, w1):
    return pl.pallas_call(
        _tc_mm_body,
        grid=(_N // _R,),
        in_specs=[
            pl.BlockSpec((_R, 128), lambda i: (i, 0)),
            pl.BlockSpec((128, 128), lambda i: (0, 0)),
        ],
        out_specs=pl.BlockSpec((_R, 128), lambda i: (i, 0)),
        out_shape=jax.ShapeDtypeStruct((_N, 128), jnp.float32),
    )(x, w1)


def _tc_norm(u, degp):
    return pl.pallas_call(
        _tc_norm_body,
        grid=(_N // _R,),
        in_specs=[
            pl.BlockSpec((_R, 128), lambda i: (i, 0)),
            pl.BlockSpec((_NC, _R, 16), lambda i: (0, i, 0)),
        ],
        out_specs=[
            pl.BlockSpec((_R, 128), lambda i: (i, 0)),
            pl.BlockSpec((_R, 128), lambda i: (i, 0)),
        ],
        out_shape=[
            jax.ShapeDtypeStruct((_N, 128), jnp.float32),
            jax.ShapeDtypeStruct((_N, 128), jnp.float32),
        ],
    )(u, degp)


def _tc_mid(p, normb, b, w, d_out):
    return pl.pallas_call(
        _tc_mid_body,
        grid=(_N // _R,),
        in_specs=[
            pl.BlockSpec((_NC, _R, 128), lambda i: (0, i, 0)),
            pl.BlockSpec((_R, 128), lambda i: (i, 0)),
            pl.BlockSpec((1, 128), lambda i: (0, 0)),
            pl.BlockSpec((128, d_out), lambda i: (0, 0)),
        ],
        out_specs=pl.BlockSpec((_R, d_out), lambda i: (i, 0)),
        out_shape=jax.ShapeDtypeStruct((_N, d_out), jnp.float32),
    )(p, normb, b, w)


def _tc_scale(p, normb, b):
    return pl.pallas_call(
        _tc_scale_body,
        grid=(_N // _R,),
        in_specs=[
            pl.BlockSpec((_NC, _R, 128), lambda i: (0, i, 0)),
            pl.BlockSpec((_R, 128), lambda i: (i, 0)),
            pl.BlockSpec((1, 128), lambda i: (0, 0)),
        ],
        out_specs=pl.BlockSpec((_R, 128), lambda i: (i, 0)),
        out_shape=jax.ShapeDtypeStruct((_N, 128), jnp.float32),
    )(p, normb, b)


def _tc_final(p, normb, b, w):
    return pl.pallas_call(
        _tc_final_body,
        grid=(_N // _R,),
        in_specs=[
            pl.BlockSpec((_NC, _R, 128), lambda i: (0, i, 0)),
            pl.BlockSpec((_R, 128), lambda i: (i, 0)),
            pl.BlockSpec((1, 64), lambda i: (0, 0)),
            pl.BlockSpec((128, 64), lambda i: (0, 0)),
        ],
        out_specs=pl.BlockSpec((_R, 64), lambda i: (i, 0)),
        out_shape=jax.ShapeDtypeStruct((_N, 64), jnp.float32),
    )(p, normb, b, w)


def kernel(features, edge_index, W1, b1, W2, b2, W3, b3):
    src = edge_index[0].reshape(_NW, _NPAN, _CPP, _K)
    dst = edge_index[1].reshape(_NW, _NPAN, _CPP, _K)
    dst2 = edge_index[1].reshape(_NW, _NCH, _K)
    ones16 = jnp.ones((_K, 16), jnp.float32)
    z16 = jnp.zeros((_RPT, 16), jnp.float32)
    z128 = jnp.zeros((_RPT, 128), jnp.float32)

    degp = _deg_kernel(dst2, ones16, z16)
    u1 = _tc_mm(features, W1)
    y1, normb = _tc_norm(u1, degp)
    p1 = _agg128(y1, src, dst, z128)
    y2 = _tc_mid(p1, normb, b1.reshape(1, 128), W2, 128)
    p2 = _agg128(y2, src, dst, z128)
    y3 = _tc_scale(p2, normb, b2.reshape(1, 128))
    p3 = _agg128(y3, src, dst, z128)
    return _tc_final(p3, normb, b3.reshape(1, 64), W3)
